# Initial kernel scaffold; baseline (speedup 1.0000x reference)
import jax
import jax.numpy as jnp
from jax.experimental import pallas as pl


def kernel(pc, table, centers, sigmas, W1, W2, W_att, W_feat, gamma, beta, attr_idx, nbr):
    raise NotImplementedError("write your pallas kernel here")


# speedup vs baseline: 1.7057x; 1.7057x over previous
import functools

import jax
import jax.numpy as jnp
from jax import lax
from jax.experimental import pallas as pl
from jax.experimental.pallas import tpu as pltpu
from jax.experimental.pallas import tpu_sc as plsc

N = 50000
K = 16
NG = 32
DE = 12
DF = 64
DP = 64
NCATP = 48

CHUNK = 128
NTILES = 32
CPT = 13
NB = CPT * CHUNK
NP = NTILES * NB
BN = 512
NBLK = NP // BN


def _sc_gather_body(pcpack_hbm, nbrt_hbm, out_hbm, idx_v, rows_v, sem):
    info = plsc.get_sparse_core_info()
    nc = info.num_cores
    wid = lax.axis_index("s") * nc + lax.axis_index("c")
    base = wid * NB

    def kstep(k, carry):
        pltpu.sync_copy(nbrt_hbm.at[k, wid], idx_v)
        for c in range(CPT):
            pltpu.async_copy(
                pcpack_hbm.at[idx_v.at[c]],
                rows_v.at[pl.ds(c * CHUNK, CHUNK)],
                sem,
            ).wait()
        pltpu.sync_copy(rows_v.at[:, 0:4], out_hbm.at[k, pl.ds(base, NB)])
        return carry

    lax.fori_loop(0, K, kstep, 0)


def _sc_gather(pcpack, nbrt3):
    mesh = plsc.VectorSubcoreMesh(core_axis_name="c", subcore_axis_name="s")
    kfn = functools.partial(
        pl.kernel,
        mesh=mesh,
        out_type=jax.ShapeDtypeStruct((K, NP, 4), jnp.float32),
        scratch_types=[
            pltpu.VMEM((CPT, CHUNK), jnp.int32),
            pltpu.VMEM((NB, 16), jnp.float32),
            pltpu.SemaphoreType.DMA,
        ],
        compiler_params=pltpu.CompilerParams(use_tc_tiling_on_sc=False),
    )(_sc_gather_body)
    return kfn(pcpack, nbrt3)


def _tc2_body(pcnb_ref, pct_ref, cen_ref, rsig_ref, tabt_ref, w1t_ref,
              w2t_ref, watt_ref, wft_ref, pooled_ref, sum_ref, sq_ref,
              ys_scr, lt_scr, acc_scr, acc2_scr):
    blk = pl.program_id(0)
    cen = cen_ref[...]
    rsig = rsig_ref[...]
    twt = jnp.dot(w2t_ref[...], tabt_ref[...],
                  preferred_element_type=jnp.float32)
    w1catt = jnp.concatenate([w1t_ref[...], twt], axis=1)
    watt = watt_ref[...]
    px = pct_ref[0:1, :]
    py = pct_ref[1:2, :]
    pz = pct_ref[2:3, :]
    cat_iota = lax.broadcasted_iota(jnp.int32, (NCATP, BN), 0).astype(
        jnp.float32)

    for k in range(K):
        X = pcnb_ref[k, 0].reshape(1, BN)
        Y = pcnb_ref[k, 1].reshape(1, BN)
        Z = pcnb_ref[k, 2].reshape(1, BN)
        A = pcnb_ref[k, 3].reshape(1, BN)
        rx = X - px
        ry = Y - py
        rz = Z - pz
        dd = jnp.sqrt(rx * rx + ry * ry + rz * rz + 1e-6)
        t = (dd - cen) * rsig
        g = jnp.exp(-0.5 * (t * t))
        oh = (A == cat_iota).astype(jnp.float32)
        h = jnp.concatenate([g, oh], axis=0)
        y = jnp.maximum(jnp.dot(w1catt, h,
                                preferred_element_type=jnp.float32), 0.0)
        ys_scr[:, k * BN:(k + 1) * BN] = y
        lt_scr[k:k + 1, :] = jnp.dot(watt, y,
                                     preferred_element_type=jnp.float32)

    logits = lt_scr[...]
    m = jnp.max(logits, axis=0, keepdims=True)
    e = jnp.exp(logits - m)
    a = e / jnp.sum(e, axis=0, keepdims=True)

    ybar = jnp.zeros((DF, BN), jnp.float32)
    for k in range(K):
        ybar = ybar + a[k:k + 1, :] * ys_scr[:, k * BN:(k + 1) * BN]
    pooled = jnp.dot(wft_ref[...], ybar,
                     preferred_element_type=jnp.float32)

    cols = blk * BN + lax.broadcasted_iota(jnp.int32, (1, BN), 1)
    msk = (cols < N).astype(jnp.float32)
    pm = pooled * msk
    pooled_ref[...] = pm

    @pl.when(blk == 0)
    def _():
        acc_scr[...] = jnp.zeros_like(acc_scr)
        acc2_scr[...] = jnp.zeros_like(acc2_scr)

    acc_scr[...] += pm
    acc2_scr[...] += pm * pm

    @pl.when(blk == NBLK - 1)
    def _():
        sum_ref[...] = jnp.broadcast_to(
            jnp.sum(acc_scr[...], axis=1, keepdims=True), (DP, 128))
        sq_ref[...] = jnp.broadcast_to(
            jnp.sum(acc2_scr[...], axis=1, keepdims=True), (DP, 128))


def _tc2_main(pcnb, pct, cen, rsig, tabt, w1t, w2t, watt, wft):
    return pl.pallas_call(
        _tc2_body,
        grid=(NBLK,),
        in_specs=[
            pl.BlockSpec((K, 4, BN), lambda i: (0, 0, i)),
            pl.BlockSpec((8, BN), lambda i: (0, i)),
            pl.BlockSpec((NG, 1), lambda i: (0, 0)),
            pl.BlockSpec((NG, 1), lambda i: (0, 0)),
            pl.BlockSpec((DE, NCATP), lambda i: (0, 0)),
            pl.BlockSpec((DF, NG), lambda i: (0, 0)),
            pl.BlockSpec((DF, DE), lambda i: (0, 0)),
            pl.BlockSpec((1, DF), lambda i: (0, 0)),
            pl.BlockSpec((DP, DF), lambda i: (0, 0)),
        ],
        out_specs=[
            pl.BlockSpec((DP, BN), lambda i: (0, i)),
            pl.BlockSpec((DP, 128), lambda i: (0, 0)),
            pl.BlockSpec((DP, 128), lambda i: (0, 0)),
        ],
        out_shape=[
            jax.ShapeDtypeStruct((DP, NP), jnp.float32),
            jax.ShapeDtypeStruct((DP, 128), jnp.float32),
            jax.ShapeDtypeStruct((DP, 128), jnp.float32),
        ],
        scratch_shapes=[
            pltpu.VMEM((DF, K * BN), jnp.float32),
            pltpu.VMEM((K, BN), jnp.float32),
            pltpu.VMEM((DP, BN), jnp.float32),
            pltpu.VMEM((DP, BN), jnp.float32),
        ],
    )(pcnb, pct, cen, rsig, tabt, w1t, w2t, watt, wft)


def _tc2_bn_body(pooled_ref, sum_ref, sq_ref, gam_ref, bet_ref, out_ref):
    mean = sum_ref[:, 0:1] / N
    var = sq_ref[:, 0:1] / N - mean * mean
    inv = lax.rsqrt(var + 1e-5)
    p = pooled_ref[...]
    o = jnp.maximum((p - mean) * inv * gam_ref[...] + bet_ref[...], 0.0)
    out_ref[...] = o.T


def _tc2_bn(pooled, s, sq, gam, bet):
    return pl.pallas_call(
        _tc2_bn_body,
        grid=(NBLK,),
        in_specs=[
            pl.BlockSpec((DP, BN), lambda i: (0, i)),
            pl.BlockSpec((DP, 128), lambda i: (0, 0)),
            pl.BlockSpec((DP, 128), lambda i: (0, 0)),
            pl.BlockSpec((DP, 1), lambda i: (0, 0)),
            pl.BlockSpec((DP, 1), lambda i: (0, 0)),
        ],
        out_specs=pl.BlockSpec((BN, DP), lambda i: (i, 0)),
        out_shape=jax.ShapeDtypeStruct((NP, DP), jnp.float32),
    )(pooled, s, sq, gam, bet)


def kernel(pc, table, centers, sigmas, W1, W2, W_att, W_feat, gamma, beta,
           attr_idx, nbr):
    aval = attr_idx.astype(jnp.float32)
    pcpack = jnp.concatenate([pc, aval[:, None]], axis=1)
    pcpack = jnp.pad(pcpack, ((0, NP - N), (0, 12)))

    nbrt = jnp.pad(nbr.astype(jnp.int32).T, ((0, 0), (0, NP - N)))
    nbrt3 = nbrt.reshape(K, NTILES, CPT, CHUNK)

    pcnb = _sc_gather(pcpack, nbrt3)
    pcnb_cm = jnp.transpose(pcnb, (0, 2, 1))

    pct = jnp.pad(pcpack[:, 0:4].T, ((0, 4), (0, 0)))
    tabp = jnp.pad(table, ((0, NCATP - table.shape[0]), (0, 0)))

    pooled, s, sq = _tc2_main(
        pcnb_cm, pct, centers.reshape(NG, 1), (1.0 / sigmas).reshape(NG, 1),
        tabp.T, W1.T, W2.T, W_att.reshape(1, DF), W_feat.T)

    out = _tc2_bn(pooled, s, sq, gamma.reshape(DP, 1), beta.reshape(DP, 1))
    return out[:N]
